# Initial kernel scaffold; baseline (speedup 1.0000x reference)
import functools

import jax
import jax.numpy as jnp
from jax import lax
from jax.experimental import pallas as pl
from jax.experimental.pallas import tpu as pltpu
from jax.experimental.pallas import tpu_sc as plsc

N = 10000
D = 128
H = 64
E = 320000
NSUB = 16
NCORE = 2
CHUNK = 128
CH_PER_SUB = 157
E_PAD = NSUB * CH_PER_SUB * CHUNK
N_PAD = 10016
ROWS_PER_SUB = N_PAD // NSUB

_mesh = plsc.VectorSubcoreMesh(core_axis_name="c", subcore_axis_name="s")


@functools.partial(
    pl.kernel,
    out_type=jax.ShapeDtypeStruct((N_PAD, 16), jnp.float32),
    mesh=_mesh,
    scratch_types=[
        pltpu.VMEM((CHUNK,), jnp.int32),
        pltpu.VMEM((CHUNK, 16), jnp.float32),
        pltpu.VMEM_SHARED((N_PAD, 16), jnp.float32),
    ],
)
def _deg_kernel(col_hbm, ones_hbm, zeros_hbm, out_hbm, colv, ones_v, deg_sh):
    c = lax.axis_index("c")
    s = lax.axis_index("s")

    @pl.when(c == 0)
    def _():
        pltpu.sync_copy(ones_hbm, ones_v)
        r0 = s * ROWS_PER_SUB
        pltpu.sync_copy(zeros_hbm.at[pl.ds(r0, ROWS_PER_SUB), :],
                        deg_sh.at[pl.ds(r0, ROWS_PER_SUB), :])
        plsc.subcore_barrier()

        def chunk(i, carry):
            base = (s * CH_PER_SUB + i) * CHUNK
            pltpu.sync_copy(col_hbm.at[pl.ds(base, CHUNK)], colv)
            pltpu.sync_copy(ones_v, deg_sh.at[colv], add=True)
            return carry

        lax.fori_loop(0, CH_PER_SUB, chunk, 0)
        plsc.subcore_barrier()
        pltpu.sync_copy(deg_sh.at[pl.ds(r0, ROWS_PER_SUB), :],
                        out_hbm.at[pl.ds(r0, ROWS_PER_SUB), :])


@functools.partial(
    pl.kernel,
    out_type=jax.ShapeDtypeStruct((NCORE, N_PAD, H), jnp.float32),
    mesh=_mesh,
    scratch_types=[
        pltpu.VMEM((CHUNK,), jnp.int32),
        pltpu.VMEM((CHUNK,), jnp.int32),
        pltpu.VMEM((CHUNK, H), jnp.float32),
        pltpu.VMEM_SHARED((N_PAD, H), jnp.float32),
        pltpu.SemaphoreType.DMA,
    ],
)
def _round_kernel(u_hbm, row_hbm, col_hbm, acc_hbm,
                  rowv, colv, rows_v, acc_sh, sem):
    c = lax.axis_index("c")
    s = lax.axis_index("s")
    u_c = u_hbm.at[c]
    r0 = s * ROWS_PER_SUB
    pltpu.sync_copy(u_c.at[pl.ds(r0, ROWS_PER_SUB), :],
                    acc_sh.at[pl.ds(r0, ROWS_PER_SUB), :])
    plsc.subcore_barrier()

    def chunk(i, carry):
        base = (s * CH_PER_SUB + i) * CHUNK
        pltpu.sync_copy(row_hbm.at[pl.ds(base, CHUNK)], rowv)
        pltpu.sync_copy(col_hbm.at[pl.ds(base, CHUNK)], colv)
        pltpu.async_copy(u_c.at[rowv], rows_v, sem).wait()
        pltpu.sync_copy(rows_v, acc_sh.at[colv], add=True)
        return carry

    lax.fori_loop(0, CH_PER_SUB, chunk, 0)
    plsc.subcore_barrier()
    pltpu.sync_copy(acc_sh.at[pl.ds(r0, ROWS_PER_SUB), :],
                    acc_hbm.at[c].at[pl.ds(r0, ROWS_PER_SUB), :])


def _prep_body(x_ref, deg_ref, u0_ref, dinv_ref, dinv2_ref):
    deg = deg_ref[:, 0:1] + 1.0
    rows = lax.broadcasted_iota(jnp.int32, (N_PAD, 1), 0)
    dinv = jnp.where(rows < N, lax.rsqrt(deg), 0.0)
    u0 = dinv * x_ref[...]
    u0_ref[0] = u0[:, :H]
    u0_ref[1] = u0[:, H:]
    dinv_ref[...] = dinv
    dinv2_ref[...] = dinv * dinv


def _prep(x_pad, deg2d):
    return pl.pallas_call(
        _prep_body,
        out_shape=(
            jax.ShapeDtypeStruct((NCORE, N_PAD, H), jnp.float32),
            jax.ShapeDtypeStruct((N_PAD, 1), jnp.float32),
            jax.ShapeDtypeStruct((N_PAD, 1), jnp.float32),
        ),
    )(x_pad, deg2d)


def _scale_body(acc_ref, dinv2_ref, u_ref):
    u_ref[...] = acc_ref[...] * dinv2_ref[...][None]


def _scale(acc, dinv2):
    return pl.pallas_call(
        _scale_body,
        out_shape=jax.ShapeDtypeStruct((NCORE, N_PAD, H), jnp.float32),
    )(acc, dinv2)


def _final_body(acc_ref, dinv_ref, wt_ref, b_ref, out_ref):
    h = jnp.concatenate([acc_ref[0], acc_ref[1]], axis=1) * dinv_ref[...]
    h = lax.slice(h, (0, 0), (N, D))
    out_ref[...] = jnp.dot(h, wt_ref[...],
                           preferred_element_type=jnp.float32) + b_ref[...]


def _final(acc, dinv, w_t, b2d):
    return pl.pallas_call(
        _final_body,
        out_shape=jax.ShapeDtypeStruct((N, D), jnp.float32),
    )(acc, dinv, w_t, b2d)


def kernel(x, edge_index, W, b):
    row = edge_index[0].astype(jnp.int32)
    col = edge_index[1].astype(jnp.int32)
    pad = E_PAD - row.shape[0]
    row_p = jnp.concatenate([row, jnp.full((pad,), N, jnp.int32)])
    col_p = jnp.concatenate([col, jnp.full((pad,), N, jnp.int32)])
    x_pad = jnp.pad(x, ((0, N_PAD - N), (0, 0)))
    ones_chunk = jnp.ones((CHUNK, 16), jnp.float32)
    zeros_init = jnp.zeros((N_PAD, 16), jnp.float32)

    deg2d = _deg_kernel(col_p, ones_chunk, zeros_init)
    u0, dinv, dinv2 = _prep(x_pad, deg2d)
    acc1 = _round_kernel(u0, row_p, col_p)
    u1 = _scale(acc1, dinv2)
    acc2 = _round_kernel(u1, row_p, col_p)
    return _final(acc2, dinv, W.T, b.reshape(1, D))


# speedup vs baseline: 11.3573x; 11.3573x over previous
import functools

import jax
import jax.numpy as jnp
from jax import lax
from jax.experimental import pallas as pl
from jax.experimental.pallas import tpu as pltpu
from jax.experimental.pallas import tpu_sc as plsc

N = 10000
D = 128
E = 320000
NSUB = 16
NCORE = 2
NW = NCORE * NSUB
CHUNK = 128
CH_PER_SUB = 79
E_PAD = NW * CH_PER_SUB * CHUNK
N_PAD = 10112
ROWS_PER_SUB = N_PAD // NSUB

_mesh = plsc.VectorSubcoreMesh(core_axis_name="c", subcore_axis_name="s")


@functools.partial(
    pl.kernel,
    out_type=jax.ShapeDtypeStruct((NCORE, N_PAD, D), jnp.float32),
    mesh=_mesh,
    scratch_types=[
        pltpu.VMEM((CHUNK,), jnp.int32),
        pltpu.VMEM((CHUNK, D), jnp.float32),
        pltpu.VMEM_SHARED((N_PAD, D), jnp.float32),
    ],
)
def _deg_kernel(col_hbm, ones_hbm, zeros_hbm, out_hbm, colv, ones_v, deg_sh):
    c = lax.axis_index("c")
    s = lax.axis_index("s")
    w = c * NSUB + s
    r0 = s * ROWS_PER_SUB
    pltpu.sync_copy(ones_hbm, ones_v)
    pltpu.sync_copy(zeros_hbm.at[pl.ds(r0, ROWS_PER_SUB), :],
                    deg_sh.at[pl.ds(r0, ROWS_PER_SUB), :])
    plsc.subcore_barrier()

    def chunk(i, carry):
        base = (w * CH_PER_SUB + i) * CHUNK
        pltpu.sync_copy(col_hbm.at[pl.ds(base, CHUNK)], colv)
        pltpu.sync_copy(ones_v, deg_sh.at[colv], add=True)
        return carry

    lax.fori_loop(0, CH_PER_SUB, chunk, 0)
    plsc.subcore_barrier()
    pltpu.sync_copy(deg_sh.at[pl.ds(r0, ROWS_PER_SUB), :],
                    out_hbm.at[c].at[pl.ds(r0, ROWS_PER_SUB), :])


@functools.partial(
    pl.kernel,
    out_type=jax.ShapeDtypeStruct((NCORE, N_PAD, D), jnp.float32),
    mesh=_mesh,
    scratch_types=[
        pltpu.VMEM((CHUNK,), jnp.int32),
        pltpu.VMEM((CHUNK,), jnp.int32),
        pltpu.VMEM((CHUNK, D), jnp.float32),
        pltpu.VMEM_SHARED((N_PAD, D), jnp.float32),
        pltpu.SemaphoreType.DMA,
    ],
)
def _round_kernel(u_hbm, row_hbm, col_hbm, zeros_hbm, acc_hbm,
                  rowv, colv, rows_v, acc_sh, sem):
    c = lax.axis_index("c")
    s = lax.axis_index("s")
    w = c * NSUB + s
    r0 = s * ROWS_PER_SUB

    @pl.when(c == 0)
    def _():
        pltpu.sync_copy(u_hbm.at[pl.ds(r0, ROWS_PER_SUB), :],
                        acc_sh.at[pl.ds(r0, ROWS_PER_SUB), :])

    @pl.when(c == 1)
    def _():
        pltpu.sync_copy(zeros_hbm.at[pl.ds(r0, ROWS_PER_SUB), :],
                        acc_sh.at[pl.ds(r0, ROWS_PER_SUB), :])

    plsc.subcore_barrier()

    def chunk(i, carry):
        base = (w * CH_PER_SUB + i) * CHUNK
        pltpu.sync_copy(row_hbm.at[pl.ds(base, CHUNK)], rowv)
        pltpu.sync_copy(col_hbm.at[pl.ds(base, CHUNK)], colv)
        pltpu.async_copy(u_hbm.at[rowv], rows_v, sem).wait()
        pltpu.sync_copy(rows_v, acc_sh.at[colv], add=True)
        return carry

    lax.fori_loop(0, CH_PER_SUB, chunk, 0)
    plsc.subcore_barrier()
    pltpu.sync_copy(acc_sh.at[pl.ds(r0, ROWS_PER_SUB), :],
                    acc_hbm.at[c].at[pl.ds(r0, ROWS_PER_SUB), :])


def _prep_body(x_ref, deg_ref, u0_ref, dinv_ref, dinv2_ref):
    deg = deg_ref[0, :, 0:1] + deg_ref[1, :, 0:1] + 1.0
    rows = lax.broadcasted_iota(jnp.int32, (N_PAD, 1), 0)
    dinv = jnp.where(rows < N, lax.rsqrt(deg), 0.0)
    u0_ref[...] = dinv * x_ref[...]
    dinv_ref[...] = dinv
    dinv2_ref[...] = dinv * dinv


def _prep(x_pad, deg2d):
    return pl.pallas_call(
        _prep_body,
        out_shape=(
            jax.ShapeDtypeStruct((N_PAD, D), jnp.float32),
            jax.ShapeDtypeStruct((N_PAD, 1), jnp.float32),
            jax.ShapeDtypeStruct((N_PAD, 1), jnp.float32),
        ),
    )(x_pad, deg2d)


def _scale_body(acc_ref, dinv2_ref, u_ref):
    u_ref[...] = (acc_ref[0] + acc_ref[1]) * dinv2_ref[...]


def _scale(acc, dinv2):
    return pl.pallas_call(
        _scale_body,
        out_shape=jax.ShapeDtypeStruct((N_PAD, D), jnp.float32),
    )(acc, dinv2)


def _final_body(acc_ref, dinv_ref, wt_ref, b_ref, out_ref):
    h = (acc_ref[0] + acc_ref[1]) * dinv_ref[...]
    h = lax.slice(h, (0, 0), (N, D))
    out_ref[...] = jnp.dot(h, wt_ref[...],
                           preferred_element_type=jnp.float32) + b_ref[...]


def _final(acc, dinv, w_t, b2d):
    return pl.pallas_call(
        _final_body,
        out_shape=jax.ShapeDtypeStruct((N, D), jnp.float32),
    )(acc, dinv, w_t, b2d)


def kernel(x, edge_index, W, b):
    row = edge_index[0].astype(jnp.int32)
    col = edge_index[1].astype(jnp.int32)
    pad = E_PAD - row.shape[0]
    row_p = jnp.concatenate([row, jnp.full((pad,), N, jnp.int32)])
    col_p = jnp.concatenate([col, jnp.full((pad,), N, jnp.int32)])
    x_pad = jnp.pad(x, ((0, N_PAD - N), (0, 0)))
    ones_chunk = jnp.ones((CHUNK, D), jnp.float32)
    zeros128 = jnp.zeros((N_PAD, D), jnp.float32)

    deg2d = _deg_kernel(col_p, ones_chunk, zeros128)
    u0, dinv, dinv2 = _prep(x_pad, deg2d)
    acc1 = _round_kernel(u0, row_p, col_p, zeros128)
    u1 = _scale(acc1, dinv2)
    acc2 = _round_kernel(u1, row_p, col_p, zeros128)
    return _final(acc2, dinv, W.T, b.reshape(1, D))
